# baseline (speedup 1.0000x reference)
import functools

import jax
import jax.numpy as jnp
from jax import lax
from jax.experimental import pallas as pl
from jax.experimental.pallas import tpu as pltpu
from jax.experimental.pallas import tpu_sc as plsc

NUM_SPARSE = 26
NUM_DENSE = 13
VOCAB_ROWS = 100000
LATENT = 16
BATCH = 16384

NUM_CORES = 2
NUM_SUBCORES = 16
NW = NUM_CORES * NUM_SUBCORES
BPW = BATCH // NW
CHUNK = 128
NCHUNK = BPW // CHUNK
GROUP = NUM_SPARSE // 2


def _sc_body(idx_hbm, *refs):
    embs = refs[0:NUM_SPARSE]
    lins = refs[NUM_SPARSE:2 * NUM_SPARSE]
    s_hbm, q_hbm, l_hbm = refs[2 * NUM_SPARSE:2 * NUM_SPARSE + 3]
    (idx_v, buf_a, buf_b, lbuf, s_v, q_v, l_v,
     sem_a, sem_b, sem_c) = refs[2 * NUM_SPARSE + 3:]

    cid = lax.axis_index("c")
    sid = lax.axis_index("s")
    wid = sid * NUM_CORES + cid
    base = wid * BPW

    pltpu.sync_copy(idx_hbm.at[wid], idx_v)

    zero = jnp.zeros((LATENT,), jnp.float32)

    @pl.loop(0, BPW)
    def _(r):
        s_v[r] = zero
        q_v[r] = zero

    @pl.loop(0, BPW // LATENT)
    def _(jj):
        l_v[pl.ds(jj * LATENT, LATENT)] = zero

    def accum_emb(buf, row_base):
        @pl.loop(0, GROUP * CHUNK)
        def _(rr):
            v = buf[rr]
            r = row_base + (rr & (CHUNK - 1))
            plsc.addupdate(s_v.at[r], v)
            plsc.addupdate(q_v.at[r], v * v)

    def accum_lin(row_base):
        @pl.loop(0, NUM_SPARSE)
        def _(t):
            @pl.loop(0, CHUNK // LATENT)
            def _(jj):
                seg = pl.ds(jj * LATENT, LATENT)
                dst = pl.ds(row_base + jj * LATENT, LATENT)
                plsc.addupdate(l_v.at[dst], lbuf[t, seg])

    @pl.loop(0, NCHUNK)
    def _(j):
        row_base = j * CHUNK
        cps_a = [
            pltpu.async_copy(embs[t].at[idx_v.at[t, j]],
                             buf_a.at[pl.ds(t * CHUNK, CHUNK)], sem_a)
            for t in range(GROUP)
        ]
        cps_b = [
            pltpu.async_copy(embs[GROUP + t].at[idx_v.at[GROUP + t, j]],
                             buf_b.at[pl.ds(t * CHUNK, CHUNK)], sem_b)
            for t in range(GROUP)
        ]
        cps_c = [
            pltpu.async_copy(lins[t].at[idx_v.at[t, j]], lbuf.at[t], sem_c)
            for t in range(NUM_SPARSE)
        ]
        for c in cps_a:
            c.wait()
        accum_emb(buf_a, row_base)
        for c in cps_b:
            c.wait()
        accum_emb(buf_b, row_base)
        for c in cps_c:
            c.wait()
        accum_lin(row_base)

    pltpu.sync_copy(s_v, s_hbm.at[pl.ds(base, BPW)])
    pltpu.sync_copy(q_v, q_hbm.at[pl.ds(base, BPW)])
    pltpu.sync_copy(l_v, l_hbm.at[pl.ds(base, BPW)])


_sc_gather = functools.partial(
    pl.kernel,
    out_type=[
        jax.ShapeDtypeStruct((BATCH, LATENT), jnp.float32),
        jax.ShapeDtypeStruct((BATCH, LATENT), jnp.float32),
        jax.ShapeDtypeStruct((BATCH,), jnp.float32),
    ],
    mesh=plsc.VectorSubcoreMesh(core_axis_name="c", subcore_axis_name="s"),
    scratch_types=[
        pltpu.VMEM((NUM_SPARSE, NCHUNK, CHUNK), jnp.int32),
        pltpu.VMEM((GROUP * CHUNK, LATENT), jnp.float32),
        pltpu.VMEM((GROUP * CHUNK, LATENT), jnp.float32),
        pltpu.VMEM((NUM_SPARSE, CHUNK), jnp.float32),
        pltpu.VMEM((BPW, LATENT), jnp.float32),
        pltpu.VMEM((BPW, LATENT), jnp.float32),
        pltpu.VMEM((BPW,), jnp.float32),
        pltpu.SemaphoreType.DMA,
        pltpu.SemaphoreType.DMA,
        pltpu.SemaphoreType.DMA,
    ],
    compiler_params=pltpu.CompilerParams(use_tc_tiling_on_sc=False),
)(_sc_body)


BM = 2048


def _tc_body(dense_ref, s_ref, q_ref, l_ref, daw_ref, dab_ref, lw_ref,
             lb_ref, bias_ref, out_ref):
    d = dense_ref[...]
    demb = jnp.dot(d, daw_ref[...],
                   preferred_element_type=jnp.float32) + dab_ref[...]
    s = s_ref[...] + demb
    q = q_ref[...] + demb * demb
    second = 0.5 * (jnp.sum(s * s, axis=1) - jnp.sum(q, axis=1))
    first = (jnp.dot(d, lw_ref[...], preferred_element_type=jnp.float32)[:, 0]
             + lb_ref[0, 0] + l_ref[...][:, 0])
    out_ref[...] = (first + second + bias_ref[0, 0])[:, None]


def _tc_combine(dense, s, q, l, daw, dab, lw, lb, bias):
    grid = BATCH // BM
    return pl.pallas_call(
        _tc_body,
        grid=(grid,),
        in_specs=[
            pl.BlockSpec((BM, NUM_DENSE), lambda i: (i, 0)),
            pl.BlockSpec((BM, LATENT), lambda i: (i, 0)),
            pl.BlockSpec((BM, LATENT), lambda i: (i, 0)),
            pl.BlockSpec((BM, 1), lambda i: (i, 0)),
            pl.BlockSpec((NUM_DENSE, LATENT), lambda i: (0, 0)),
            pl.BlockSpec((1, LATENT), lambda i: (0, 0)),
            pl.BlockSpec((NUM_DENSE, 1), lambda i: (0, 0)),
            pl.BlockSpec((1, 1), lambda i: (0, 0)),
            pl.BlockSpec((1, 1), lambda i: (0, 0)),
        ],
        out_specs=pl.BlockSpec((BM, 1), lambda i: (i, 0)),
        out_shape=jax.ShapeDtypeStruct((BATCH, 1), jnp.float32),
    )(dense, s, q, l, daw, dab, lw, lb, bias)


def kernel(dense_0, dense_1, dense_2, dense_3, dense_4, dense_5, dense_6, dense_7, dense_8, dense_9, dense_10, dense_11, dense_12, sparse_0, sparse_1, sparse_2, sparse_3, sparse_4, sparse_5, sparse_6, sparse_7, sparse_8, sparse_9, sparse_10, sparse_11, sparse_12, sparse_13, sparse_14, sparse_15, sparse_16, sparse_17, sparse_18, sparse_19, sparse_20, sparse_21, sparse_22, sparse_23, sparse_24, sparse_25, lin_table_0, lin_table_1, lin_table_2, lin_table_3, lin_table_4, lin_table_5, lin_table_6, lin_table_7, lin_table_8, lin_table_9, lin_table_10, lin_table_11, lin_table_12, lin_table_13, lin_table_14, lin_table_15, lin_table_16, lin_table_17, lin_table_18, lin_table_19, lin_table_20, lin_table_21, lin_table_22, lin_table_23, lin_table_24, lin_table_25, emb_table_0, emb_table_1, emb_table_2, emb_table_3, emb_table_4, emb_table_5, emb_table_6, emb_table_7, emb_table_8, emb_table_9, emb_table_10, emb_table_11, emb_table_12, emb_table_13, emb_table_14, emb_table_15, emb_table_16, emb_table_17, emb_table_18, emb_table_19, emb_table_20, emb_table_21, emb_table_22, emb_table_23, emb_table_24, emb_table_25, lin_dense_w, lin_dense_b, dense_arch_w, dense_arch_b, bias):
    denses = [dense_0, dense_1, dense_2, dense_3, dense_4, dense_5, dense_6,
              dense_7, dense_8, dense_9, dense_10, dense_11, dense_12]
    sparses = [sparse_0, sparse_1, sparse_2, sparse_3, sparse_4, sparse_5,
               sparse_6, sparse_7, sparse_8, sparse_9, sparse_10, sparse_11,
               sparse_12, sparse_13, sparse_14, sparse_15, sparse_16,
               sparse_17, sparse_18, sparse_19, sparse_20, sparse_21,
               sparse_22, sparse_23, sparse_24, sparse_25]
    lin_tables = [lin_table_0, lin_table_1, lin_table_2, lin_table_3,
                  lin_table_4, lin_table_5, lin_table_6, lin_table_7,
                  lin_table_8, lin_table_9, lin_table_10, lin_table_11,
                  lin_table_12, lin_table_13, lin_table_14, lin_table_15,
                  lin_table_16, lin_table_17, lin_table_18, lin_table_19,
                  lin_table_20, lin_table_21, lin_table_22, lin_table_23,
                  lin_table_24, lin_table_25]
    emb_tables = [emb_table_0, emb_table_1, emb_table_2, emb_table_3,
                  emb_table_4, emb_table_5, emb_table_6, emb_table_7,
                  emb_table_8, emb_table_9, emb_table_10, emb_table_11,
                  emb_table_12, emb_table_13, emb_table_14, emb_table_15,
                  emb_table_16, emb_table_17, emb_table_18, emb_table_19,
                  emb_table_20, emb_table_21, emb_table_22, emb_table_23,
                  emb_table_24, emb_table_25]

    idx = jnp.stack([s.astype(jnp.int32) for s in sparses], axis=0)
    idx = idx.reshape(NUM_SPARSE, NW, NCHUNK, CHUNK).transpose(1, 0, 2, 3)

    lins_flat = [t.reshape(VOCAB_ROWS) for t in lin_tables]

    s, q, l = _sc_gather(idx, *emb_tables, *lins_flat)

    dense = jnp.stack(denses, axis=1)
    out = _tc_combine(dense, s, q, l.reshape(BATCH, 1),
                      dense_arch_w, dense_arch_b.reshape(1, LATENT),
                      lin_dense_w, lin_dense_b.reshape(1, 1), bias)
    return out
